# baseline (speedup 1.0000x reference)
import jax
import jax.numpy as jnp
from jax.experimental import pallas as pl
from jax.experimental.pallas import tpu as pltpu

N = 100000
D = 128
G = 256
BLOCK = 10000
NB = N // BLOCK
GC = 128
NGC = G // GC


def _body(x_ref, ids_ref, w_ref, b_ref, q_ref, ones_ref, blo_ref, bhi_ref,
          out_ref, m_ref, d_ref, s_ref, score_ref):
    i = pl.program_id(0)
    j = pl.program_id(1)

    @pl.when((i == 0) & (j == 0))
    def _init():
        m_ref[...] = jnp.full((NGC, GC), -1e30, jnp.float32)
        d_ref[...] = jnp.zeros((NGC, GC), jnp.float32)
        s_ref[...] = jnp.zeros((NGC, D, GC), jnp.float32)

    @pl.when(j == 0)
    def _score():
        g = jnp.tanh(
            jax.lax.dot_general(
                x_ref[...], w_ref[...], (((1,), (1,)), ((), ())),
                preferred_element_type=jnp.float32,
            )
            + b_ref[...]
        )
        score_ref[...] = jax.lax.dot_general(
            g, q_ref[...], (((1,), (0,)), ((), ())),
            preferred_element_type=jnp.float32,
        )

    overlap = (blo_ref[i] < (j + 1) * GC) & (bhi_ref[i] >= j * GC)

    @pl.when(overlap)
    def _update():
        xb = x_ref[...]
        score = score_ref[...]
        ids = ids_ref[0]
        col = jax.lax.broadcasted_iota(jnp.int32, (BLOCK, GC), 1) + j * GC
        one_hot = ids == col

        masked = jnp.where(one_hot, jnp.broadcast_to(score, (BLOCK, GC)), -1e30)
        bm = jnp.max(masked, axis=0, keepdims=True)
        m_old = m_ref[pl.ds(j, 1), :]
        m_new = jnp.maximum(m_old, bm)
        scale = jnp.exp(m_old - m_new)

        e = jnp.exp(jnp.broadcast_to(score, (BLOCK, GC)) - m_new)
        p = jnp.where(one_hot, e, 0.0)

        d_ref[pl.ds(j, 1), :] = d_ref[pl.ds(j, 1), :] * scale + jax.lax.dot_general(
            ones_ref[...], p, (((1,), (0,)), ((), ())),
            preferred_element_type=jnp.float32,
        )
        s_ref[j] = s_ref[j] * scale + jax.lax.dot_general(
            xb, p, (((0,), (0,)), ((), ())), preferred_element_type=jnp.float32
        )
        m_ref[pl.ds(j, 1), :] = m_new

    @pl.when((i == NB - 1) & (j == NGC - 1))
    def _fini():
        for jj in range(NGC):
            d = d_ref[pl.ds(jj, 1), :]
            d = jnp.where(d == 0.0, 1.0, d)
            out_ref[pl.ds(jj * GC, GC), :] = (s_ref[jj] / d).T


@jax.jit
def kernel(x, graph_ptr, W, b, query):
    ids2d = graph_ptr.reshape(NB, BLOCK)
    blo = ids2d[:, 0]
    bhi = ids2d[:, -1]
    ids = ids2d.reshape(NB, BLOCK, 1)
    b2 = b.reshape(1, D)
    q2 = query.reshape(D, 1)
    ones = jnp.ones((1, BLOCK), jnp.float32)
    return pl.pallas_call(
        _body,
        grid=(NB, NGC),
        in_specs=[
            pl.BlockSpec((BLOCK, D), lambda i, j: (i, 0)),
            pl.BlockSpec((1, BLOCK, 1), lambda i, j: (i, 0, 0)),
            pl.BlockSpec((D, D), lambda i, j: (0, 0)),
            pl.BlockSpec((1, D), lambda i, j: (0, 0)),
            pl.BlockSpec((D, 1), lambda i, j: (0, 0)),
            pl.BlockSpec((1, BLOCK), lambda i, j: (0, 0)),
            pl.BlockSpec(memory_space=pltpu.SMEM),
            pl.BlockSpec(memory_space=pltpu.SMEM),
        ],
        out_specs=pl.BlockSpec((G, D), lambda i, j: (0, 0)),
        out_shape=jax.ShapeDtypeStruct((G, D), jnp.float32),
        scratch_shapes=[
            pltpu.VMEM((NGC, GC), jnp.float32),
            pltpu.VMEM((NGC, GC), jnp.float32),
            pltpu.VMEM((NGC, D, GC), jnp.float32),
            pltpu.VMEM((BLOCK, 1), jnp.float32),
        ],
    )(x, ids, W, b2, q2, ones, blo, bhi)


# speedup vs baseline: 1.2998x; 1.2998x over previous
import jax
import jax.numpy as jnp
from jax.experimental import pallas as pl
from jax.experimental.pallas import tpu as pltpu

N = 100000
D = 128
G = 256
BLOCK = 10000
NB = N // BLOCK

NEG = -1e30


def _body(x_ref, ids_ref, w_ref, b_ref, q_ref, ones_ref,
          out_ref, m_ref, d_ref, s_ref):
    i = pl.program_id(0)

    @pl.when(i == 0)
    def _init():
        m_ref[...] = jnp.full((1, G), NEG, jnp.float32)
        d_ref[...] = jnp.zeros((1, G), jnp.float32)
        s_ref[...] = jnp.zeros((D, G), jnp.float32)

    xb = x_ref[...]
    g = jnp.tanh(
        jax.lax.dot_general(
            xb, w_ref[...], (((1,), (1,)), ((), ())),
            preferred_element_type=jnp.float32,
        )
        + b_ref[...]
    )
    score = jax.lax.dot_general(
        g, q_ref[...], (((1,), (0,)), ((), ())),
        preferred_element_type=jnp.float32,
    )

    ids = ids_ref[0]
    one_hot = ids == jax.lax.broadcasted_iota(jnp.int32, (BLOCK, G), 1)

    masked = jnp.where(one_hot, jnp.broadcast_to(score, (BLOCK, G)), NEG)
    bm = jnp.max(masked, axis=0, keepdims=True)
    m_old = m_ref[...]
    m_new = jnp.maximum(m_old, bm)
    scale = jnp.exp(m_old - m_new)

    p = jnp.exp(masked - m_new).astype(jnp.bfloat16)

    d_ref[...] = d_ref[...] * scale + jax.lax.dot_general(
        ones_ref[...], p, (((1,), (0,)), ((), ())),
        preferred_element_type=jnp.float32,
    )
    s_ref[...] = s_ref[...] * scale + jax.lax.dot_general(
        xb.astype(jnp.bfloat16), p, (((0,), (0,)), ((), ())),
        preferred_element_type=jnp.float32,
    )
    m_ref[...] = m_new

    @pl.when(i == NB - 1)
    def _fini():
        d = d_ref[...]
        d = jnp.where(d == 0.0, 1.0, d)
        out_ref[...] = (s_ref[...] / d).T


@jax.jit
def kernel(x, graph_ptr, W, b, query):
    ids = graph_ptr.reshape(NB, BLOCK, 1)
    b2 = b.reshape(1, D)
    q2 = query.reshape(D, 1)
    ones = jnp.ones((1, BLOCK), jnp.bfloat16)
    return pl.pallas_call(
        _body,
        grid=(NB,),
        in_specs=[
            pl.BlockSpec((BLOCK, D), lambda i: (i, 0)),
            pl.BlockSpec((1, BLOCK, 1), lambda i: (i, 0, 0)),
            pl.BlockSpec((D, D), lambda i: (0, 0)),
            pl.BlockSpec((1, D), lambda i: (0, 0)),
            pl.BlockSpec((D, 1), lambda i: (0, 0)),
            pl.BlockSpec((1, BLOCK), lambda i: (0, 0)),
        ],
        out_specs=pl.BlockSpec((G, D), lambda i: (0, 0)),
        out_shape=jax.ShapeDtypeStruct((G, D), jnp.float32),
        scratch_shapes=[
            pltpu.VMEM((1, G), jnp.float32),
            pltpu.VMEM((1, G), jnp.float32),
            pltpu.VMEM((D, G), jnp.float32),
        ],
    )(x, ids, W, b2, q2, ones)
